# baseline (speedup 1.0000x reference)
import jax
import jax.numpy as jnp
from jax import lax
from jax.experimental import pallas as pl
from jax.experimental.pallas import tpu as pltpu
from jax.experimental.pallas import tpu_sc as plsc

_K = 2048
_B = 8
_N = 8192
_D = 512

_NBLK = 512


def _sumsq_body(x_ref, o_ref):
    x = x_ref[0]
    sq = x * x
    p = ((sq[:, 0:128] + sq[:, 128:256]) + sq[:, 256:384]) + sq[:, 384:512]
    s = p[:, 0:8]
    for t in range(1, 16):
        s = s + p[:, 8 * t:8 * t + 8]
    u = s[:, 0:4] + s[:, 4:8]
    u = u[:, 0:2] + u[:, 2:4]
    tot = u[:, 0:1] + u[:, 1:2]
    o_ref[...] = tot[:, 0][None, None, :]


def _sumsq(emb):
    out = pl.pallas_call(
        _sumsq_body,
        grid=(_B, _N // _NBLK),
        in_specs=[pl.BlockSpec((1, _NBLK, _D), lambda b, i: (b, i, 0))],
        out_specs=pl.BlockSpec((1, 1, _NBLK), lambda b, i: (b, 0, i)),
        out_shape=jax.ShapeDtypeStruct((_B, 1, _N), jnp.float32),
    )(emb)
    return out.reshape(_B, _N)


_INFO = plsc.get_sparse_core_info()
_NC, _NS = _INFO.num_cores, _INFO.num_subcores
_NW = _NC * _NS
_ROWS = _B * _K
_PER_W = _ROWS // _NW
_CH = 64


_PW = 128


def _gather_body(se_ref, te_ref, pts_ref,
                 sidx_ref, tidx_ref,
                 o_sp, o_tp, o_se, o_te,
                 idx_v, rows_v, prow_v, sem):
    wid = lax.axis_index("s") * _NC + lax.axis_index("c")
    base = wid * _PER_W
    for idx_ref, tab, out, buf in ((sidx_ref, se_ref, o_se, rows_v),
                                   (tidx_ref, te_ref, o_te, rows_v),
                                   (sidx_ref, pts_ref, o_sp, prow_v),
                                   (tidx_ref, pts_ref, o_tp, prow_v)):
        for c in range(_PER_W // _CH):
            b0 = base + c * _CH
            pltpu.sync_copy(idx_ref.at[pl.ds(b0, _CH)], idx_v)
            pltpu.async_copy(tab.at[idx_v], buf, sem).wait()
            pltpu.sync_copy(buf, out.at[pl.ds(b0, _CH)])


def _gather(se, te, pts, sidx_flat, tidx_flat):
    mesh = plsc.VectorSubcoreMesh(core_axis_name="c", subcore_axis_name="s")
    f = pl.kernel(
        _gather_body,
        mesh=mesh,
        out_type=(
            jax.ShapeDtypeStruct((_ROWS, _PW), jnp.float32),
            jax.ShapeDtypeStruct((_ROWS, _PW), jnp.float32),
            jax.ShapeDtypeStruct((_ROWS, _D), jnp.float32),
            jax.ShapeDtypeStruct((_ROWS, _D), jnp.float32),
        ),
        scratch_types=[
            pltpu.VMEM((_CH,), jnp.int32),
            pltpu.VMEM((_CH, _D), jnp.float32),
            pltpu.VMEM((_CH, _PW), jnp.float32),
            pltpu.SemaphoreType.DMA,
        ],
    )
    return f(se, te, pts, sidx_flat, tidx_flat)


def kernel(src, tgt, n0, n1, src_embedding, tgt_embedding):
    se_flat = src_embedding.reshape(_B * _N, _D)
    te_flat = tgt_embedding.reshape(_B * _N, _D)
    src_norm = jnp.sqrt(_sumsq(src_embedding))
    tgt_norm = jnp.sqrt(_sumsq(tgt_embedding))
    _, sidx = lax.top_k(src_norm, _K)
    _, tidx = lax.top_k(tgt_norm, _K)
    off = (jnp.arange(_B, dtype=jnp.int32) * _N)[:, None]
    sidx_flat = (sidx + off).reshape(_ROWS)
    tidx_flat = (tidx + off).reshape(_ROWS)
    pts = jnp.concatenate([src, n0, tgt, n1], axis=-1)
    pts = jnp.pad(pts, ((0, 0), (0, 0), (0, _PW - 12))).reshape(_B * _N, _PW)
    o_sp, o_tp, o_se, o_te = _gather(se_flat, te_flat, pts,
                                     sidx_flat, tidx_flat)
    o_sp = o_sp.reshape(_B, _K, _PW)
    o_tp = o_tp.reshape(_B, _K, _PW)
    return (o_sp[..., 0:3],
            o_tp[..., 6:9],
            o_sp[..., 3:6],
            o_tp[..., 9:12],
            o_se.reshape(_B, _K, _D),
            o_te.reshape(_B, _K, _D))


# speedup vs baseline: 1.5107x; 1.5107x over previous
import jax
import jax.numpy as jnp
from jax import lax
from jax.experimental import pallas as pl
from jax.experimental.pallas import tpu as pltpu
from jax.experimental.pallas import tpu_sc as plsc

_K = 2048
_B = 8
_N = 8192
_D = 512

_NBLK = 512


def _sumsq_body(x_ref, o_ref):
    x = x_ref[0]
    sq = x * x
    p = ((sq[:, 0:128] + sq[:, 128:256]) + sq[:, 256:384]) + sq[:, 384:512]
    p3 = p.T.reshape(16, 8, _NBLK)
    s = p3[0]
    for t in range(1, 16):
        s = s + p3[t]
    u = s[0:4] + s[4:8]
    u = u[0:2] + u[2:4]
    tot = u[0:1] + u[1:2]
    o_ref[...] = tot[None]


def _sumsq(emb):
    out = pl.pallas_call(
        _sumsq_body,
        grid=(_B, _N // _NBLK),
        in_specs=[pl.BlockSpec((1, _NBLK, _D), lambda b, i: (b, i, 0))],
        out_specs=pl.BlockSpec((1, 1, _NBLK), lambda b, i: (b, 0, i)),
        out_shape=jax.ShapeDtypeStruct((_B, 1, _N), jnp.float32),
    )(emb)
    return out.reshape(_B, _N)


_INFO = plsc.get_sparse_core_info()
_NC, _NS = _INFO.num_cores, _INFO.num_subcores
_NW = _NC * _NS
_ROWS = _B * _K
_PER_W = _ROWS // _NW
_CH = 64


_PW = 128


def _gather_body(se_ref, te_ref, pts_ref,
                 sidx_ref, tidx_ref,
                 o_sp, o_tp, o_se, o_te,
                 idx_v, rows_v, prow_v, sem):
    wid = lax.axis_index("s") * _NC + lax.axis_index("c")
    base = wid * _PER_W
    for idx_ref, tab, out, buf in ((sidx_ref, se_ref, o_se, rows_v),
                                   (tidx_ref, te_ref, o_te, rows_v),
                                   (sidx_ref, pts_ref, o_sp, prow_v),
                                   (tidx_ref, pts_ref, o_tp, prow_v)):
        for c in range(_PER_W // _CH):
            b0 = base + c * _CH
            pltpu.sync_copy(idx_ref.at[pl.ds(b0, _CH)], idx_v)
            pltpu.async_copy(tab.at[idx_v], buf, sem).wait()
            pltpu.sync_copy(buf, out.at[pl.ds(b0, _CH)])


def _gather(se, te, pts, sidx_flat, tidx_flat):
    mesh = plsc.VectorSubcoreMesh(core_axis_name="c", subcore_axis_name="s")
    f = pl.kernel(
        _gather_body,
        mesh=mesh,
        out_type=(
            jax.ShapeDtypeStruct((_ROWS, _PW), jnp.float32),
            jax.ShapeDtypeStruct((_ROWS, _PW), jnp.float32),
            jax.ShapeDtypeStruct((_ROWS, _D), jnp.float32),
            jax.ShapeDtypeStruct((_ROWS, _D), jnp.float32),
        ),
        scratch_types=[
            pltpu.VMEM((_CH,), jnp.int32),
            pltpu.VMEM((_CH, _D), jnp.float32),
            pltpu.VMEM((_CH, _PW), jnp.float32),
            pltpu.SemaphoreType.DMA,
        ],
    )
    return f(se, te, pts, sidx_flat, tidx_flat)


def kernel(src, tgt, n0, n1, src_embedding, tgt_embedding):
    se_flat = src_embedding.reshape(_B * _N, _D)
    te_flat = tgt_embedding.reshape(_B * _N, _D)
    src_norm = jnp.sqrt(_sumsq(src_embedding))
    tgt_norm = jnp.sqrt(_sumsq(tgt_embedding))
    _, sidx = lax.top_k(src_norm, _K)
    _, tidx = lax.top_k(tgt_norm, _K)
    off = (jnp.arange(_B, dtype=jnp.int32) * _N)[:, None]
    sidx_flat = (sidx + off).reshape(_ROWS)
    tidx_flat = (tidx + off).reshape(_ROWS)
    pts = jnp.concatenate([src, n0, tgt, n1], axis=-1)
    pts = jnp.pad(pts, ((0, 0), (0, 0), (0, _PW - 12))).reshape(_B * _N, _PW)
    o_sp, o_tp, o_se, o_te = _gather(se_flat, te_flat, pts,
                                     sidx_flat, tidx_flat)
    o_sp = o_sp.reshape(_B, _K, _PW)
    o_tp = o_tp.reshape(_B, _K, _PW)
    return (o_sp[..., 0:3],
            o_tp[..., 6:9],
            o_sp[..., 3:6],
            o_tp[..., 9:12],
            o_se.reshape(_B, _K, _D),
            o_te.reshape(_B, _K, _D))


# speedup vs baseline: 1.5857x; 1.0497x over previous
import jax
import jax.numpy as jnp
from jax import lax
from jax.experimental import pallas as pl
from jax.experimental.pallas import tpu as pltpu
from jax.experimental.pallas import tpu_sc as plsc

_K = 2048
_B = 8
_N = 8192
_D = 512

_NBLK = 512


def _sumsq_body(x_ref, o_ref):
    x = x_ref[0]
    sq = x * x
    p = ((sq[:, 0:128] + sq[:, 128:256]) + sq[:, 256:384]) + sq[:, 384:512]
    p3 = p.T.reshape(16, 8, _NBLK)
    s = p3[0]
    for t in range(1, 16):
        s = s + p3[t]
    u = s[0:4] + s[4:8]
    u = u[0:2] + u[2:4]
    tot = u[0:1] + u[1:2]
    o_ref[...] = tot[None]


def _sumsq(emb):
    out = pl.pallas_call(
        _sumsq_body,
        grid=(_B, _N // _NBLK),
        in_specs=[pl.BlockSpec((1, _NBLK, _D), lambda b, i: (b, i, 0))],
        out_specs=pl.BlockSpec((1, 1, _NBLK), lambda b, i: (b, 0, i)),
        out_shape=jax.ShapeDtypeStruct((_B, 1, _N), jnp.float32),
    )(emb)
    return out.reshape(_B, _N)


_INFO = plsc.get_sparse_core_info()
_NC, _NS = _INFO.num_cores, _INFO.num_subcores
_NW = _NC * _NS
_ROWS = _B * _K
_PER_W = _ROWS // _NW
_CH = 64


_PW = 128


def _topk_body(v_ref, o_ref):
    v = v_ref[...]
    lane = lax.broadcasted_iota(jnp.int32, (_B, _N), 1)
    idx = lane + _N * lax.broadcasted_iota(jnp.int32, (_B, _N), 0)
    k = 2
    while k <= _N:
        j = k // 2
        while j >= 1:
            is_lo = (lane & j) == 0
            asc = (lane & k) == 0
            vp = jnp.where(is_lo, jnp.roll(v, -j, axis=1),
                           jnp.roll(v, j, axis=1))
            ip = jnp.where(is_lo, jnp.roll(idx, -j, axis=1),
                           jnp.roll(idx, j, axis=1))
            self_first = (v > vp) | ((v == vp) & (idx < ip))
            keep = self_first ^ (asc ^ is_lo)
            v = jnp.where(keep, v, vp)
            idx = jnp.where(keep, idx, ip)
            j //= 2
        k *= 2
    o_ref[...] = idx[:, :_K]


def _topk_flat_idx(norms):
    return pl.pallas_call(
        _topk_body,
        out_shape=jax.ShapeDtypeStruct((_B, _K), jnp.int32),
    )(norms).reshape(_ROWS)


def _gather_body(se_ref, te_ref, pts_ref,
                 sidx_ref, tidx_ref,
                 o_sp, o_tp, o_se, o_te,
                 sidx_v, tidx_v, rows0, rows1, prow0, prow1,
                 gs0, gs1, ws0, ws1):
    wid = lax.axis_index("s") * _NC + lax.axis_index("c")
    base = wid * _PER_W
    pltpu.sync_copy(sidx_ref.at[pl.ds(base, _PER_W)], sidx_v)
    pltpu.sync_copy(tidx_ref.at[pl.ds(base, _PER_W)], tidx_v)
    nch = _PER_W // _CH
    tasks = []
    for idx_v, tab, out, bufs in ((sidx_v, se_ref, o_se, (rows0, rows1)),
                                  (tidx_v, te_ref, o_te, (rows0, rows1)),
                                  (sidx_v, pts_ref, o_sp, (prow0, prow1)),
                                  (tidx_v, pts_ref, o_tp, (prow0, prow1))):
        for c in range(nch):
            tasks.append((idx_v, tab, out, bufs, c))
    gsems = (gs0, gs1)
    wsems = (ws0, ws1)

    def start_gather(t, p):
        idx_v, tab, _, bufs, c = tasks[t]
        return pltpu.async_copy(tab.at[idx_v.at[pl.ds(c * _CH, _CH)]],
                                bufs[p], gsems[p])

    def start_wb(t, p):
        idx_v, _, out, bufs, c = tasks[t]
        b0 = base + c * _CH
        return pltpu.async_copy(bufs[p], out.at[pl.ds(b0, _CH)], wsems[p])

    nt = len(tasks)
    gh = [None, None]
    wh = [None, None]
    gh[0] = start_gather(0, 0)
    for t in range(nt):
        p = t % 2
        q = (t + 1) % 2
        gh[p].wait()
        if t + 1 < nt:
            if wh[q] is not None:
                wh[q].wait()
            gh[q] = start_gather(t + 1, q)
        wh[p] = start_wb(t, p)
    wh[(nt - 1) % 2].wait()


def _gather(se, te, pts, sidx_flat, tidx_flat):
    mesh = plsc.VectorSubcoreMesh(core_axis_name="c", subcore_axis_name="s")
    f = pl.kernel(
        _gather_body,
        mesh=mesh,
        out_type=(
            jax.ShapeDtypeStruct((_ROWS, _PW), jnp.float32),
            jax.ShapeDtypeStruct((_ROWS, _PW), jnp.float32),
            jax.ShapeDtypeStruct((_ROWS, _D), jnp.float32),
            jax.ShapeDtypeStruct((_ROWS, _D), jnp.float32),
        ),
        scratch_types=[
            pltpu.VMEM((_PER_W,), jnp.int32),
            pltpu.VMEM((_PER_W,), jnp.int32),
            pltpu.VMEM((_CH, _D), jnp.float32),
            pltpu.VMEM((_CH, _D), jnp.float32),
            pltpu.VMEM((_CH, _PW), jnp.float32),
            pltpu.VMEM((_CH, _PW), jnp.float32),
            pltpu.SemaphoreType.DMA,
            pltpu.SemaphoreType.DMA,
            pltpu.SemaphoreType.DMA,
            pltpu.SemaphoreType.DMA,
        ],
    )
    return f(se, te, pts, sidx_flat, tidx_flat)


def kernel(src, tgt, n0, n1, src_embedding, tgt_embedding):
    se_flat = src_embedding.reshape(_B * _N, _D)
    te_flat = tgt_embedding.reshape(_B * _N, _D)
    src_norm = jnp.sqrt(_sumsq(src_embedding))
    tgt_norm = jnp.sqrt(_sumsq(tgt_embedding))
    sidx_flat = _topk_flat_idx(src_norm)
    tidx_flat = _topk_flat_idx(tgt_norm)
    pts = jnp.concatenate([src, n0, tgt, n1], axis=-1)
    pts = jnp.pad(pts, ((0, 0), (0, 0), (0, _PW - 12))).reshape(_B * _N, _PW)
    o_sp, o_tp, o_se, o_te = _gather(se_flat, te_flat, pts,
                                     sidx_flat, tidx_flat)
    o_sp = o_sp.reshape(_B, _K, _PW)
    o_tp = o_tp.reshape(_B, _K, _PW)
    return (o_sp[..., 0:3],
            o_tp[..., 6:9],
            o_sp[..., 3:6],
            o_tp[..., 9:12],
            o_se.reshape(_B, _K, _D),
            o_te.reshape(_B, _K, _D))
